# Initial kernel scaffold; baseline (speedup 1.0000x reference)
import jax
import jax.numpy as jnp
from jax.experimental import pallas as pl


def kernel(f_in, edge_index, adj_vals, gat_Wself, gat_bself, gat_Wneigh, gat_bneigh, gat_aself, gat_aneigh, gcn_W, gcn_b, W_self_p, b_self_p, W_cls, b_cls):
    raise NotImplementedError("write your pallas kernel here")


# speedup vs baseline: 12.3912x; 12.3912x over previous
import functools

import jax
import jax.numpy as jnp
from jax import lax
from jax.experimental import pallas as pl
from jax.experimental.pallas import tpu as pltpu
from jax.experimental.pallas import tpu_sc as plsc

N = 10000
E = 320000
D = 128
HEADS = 8
HD = 16
NCLASS = 8
F32 = jnp.float32

NSUB = 16
EPT = E // NSUB
EC = 80
NCHUNK = EPT // EC
NPAD = 10240
RPT = NPAD // NSUB
ZROWS = 64
LANES = 16

_MESH = plsc.VectorSubcoreMesh(core_axis_name="c", subcore_axis_name="s")


@functools.partial(
    pl.kernel, mesh=_MESH,
    out_type=jax.ShapeDtypeStruct((NPAD, D), F32),
    scratch_types=[
        pltpu.VMEM_SHARED((NPAD, D), F32),
        pltpu.VMEM((1, EC), jnp.int32),
        pltpu.VMEM((1, EC), jnp.int32),
        pltpu.VMEM((1, EC), F32),
        pltpu.VMEM((EC, D), F32),
        pltpu.VMEM((ZROWS, D), F32),
    ],
)
def _sc_agg(table, row_h, col_h, v_h, out, acc, rowc, colc, vc, gbuf, zbuf):
    s = lax.axis_index("s")
    zero = jnp.zeros((LANES,), F32)

    @pl.loop(0, ZROWS)
    def _(r):
        zrow = zbuf.at[r]
        for k in range(D // LANES):
            zrow[pl.ds(LANES * k, LANES)] = zero

    for i in range(RPT // ZROWS):
        pltpu.sync_copy(zbuf, acc.at[pl.ds(s * RPT + i * ZROWS, ZROWS)])
    plsc.subcore_barrier()

    @pl.loop(0, NCHUNK)
    def _(j):
        g = s * NCHUNK + j
        pltpu.sync_copy(row_h.at[g], rowc)
        pltpu.sync_copy(col_h.at[g], colc)
        pltpu.sync_copy(v_h.at[g], vc)
        pltpu.sync_copy(table.at[colc.at[0]], gbuf)
        vrow = vc.at[0]

        @pl.loop(0, EC // LANES)
        def _(t):
            vt = vrow[pl.ds(LANES * t, LANES)]
            for l in range(LANES):
                vb = jnp.full((LANES,), vt[l], F32)
                grow = gbuf.at[LANES * t + l]
                for k in range(D // LANES):
                    sl = pl.ds(LANES * k, LANES)
                    grow[sl] = grow[sl] * vb

        pltpu.sync_copy(gbuf, acc.at[rowc.at[0]], add=True)

    plsc.subcore_barrier()
    pltpu.sync_copy(acc.at[pl.ds(s * RPT, RPT)], out.at[pl.ds(s * RPT, RPT)])


def _lrelu(x):
    return jnp.where(x > 0, x, 0.2 * x)


def _bn(x):
    m = jnp.mean(x, axis=0, keepdims=True)
    xc = x - m
    var = jnp.mean(xc * xc, axis=0, keepdims=True)
    return xc * lax.rsqrt(var + 1e-9)


def _dot(a, b):
    return jnp.dot(a, b, preferred_element_type=F32)


def _gat_dense(h, ws, bs, wn, bn_, ams, amn, e8):
    def body(h_r, ws_r, bs_r, wn_r, bn_r, ams_r, amn_r, e8_r, fn_o, u_o, as_o):
        hh = h_r[...]
        fs = jnp.maximum(_dot(hh, ws_r[...]) + bs_r[...], 0.0)
        fn = jnp.maximum(_dot(hh, wn_r[...]) + bn_r[...], 0.0)
        as_o[...] = _lrelu(_dot(fs, ams_r[...]))
        an = _lrelu(_dot(fs, amn_r[...]))
        fn_o[...] = fn
        u_o[...] = _dot(an, e8_r[...]) * fn

    return pl.pallas_call(
        body,
        out_shape=[jax.ShapeDtypeStruct((N, D), F32),
                   jax.ShapeDtypeStruct((N, D), F32),
                   jax.ShapeDtypeStruct((N, HEADS), F32)],
    )(h, ws, bs, wn, bn_, ams, amn, e8)


def _gat_mid(p1, q1, as1, ws, bs, wn, bn_, ams, amn, e8):
    def body(p_r, q_r, a_r, ws_r, bs_r, wn_r, bn_r, ams_r, amn_r, e8_r,
             fn_o, u_o, as_o):
        h1 = _bn(_dot(a_r[...], e8_r[...]) * p_r[...] + q_r[...])
        fs = jnp.maximum(_dot(h1, ws_r[...]) + bs_r[...], 0.0)
        fn = jnp.maximum(_dot(h1, wn_r[...]) + bn_r[...], 0.0)
        as_o[...] = _lrelu(_dot(fs, ams_r[...]))
        an = _lrelu(_dot(fs, amn_r[...]))
        fn_o[...] = fn
        u_o[...] = _dot(an, e8_r[...]) * fn

    return pl.pallas_call(
        body,
        out_shape=[jax.ShapeDtypeStruct((N, D), F32),
                   jax.ShapeDtypeStruct((N, D), F32),
                   jax.ShapeDtypeStruct((N, HEADS), F32)],
    )(p1, q1, as1, ws, bs, wn, bn_, ams, amn, e8)


def _gcn_mid(s_agg, deg16, w0, b0, w1, b1):
    def body(s_r, d_r, w0_r, b0_r, w1_r, b1_r, g0_o, g1_o, g2_o):
        ss = s_r[...]
        deg = d_r[:, 0:1]
        for b, out in enumerate((g0_o, g1_o, g2_o)):
            g1 = _bn(jnp.maximum(_dot(ss, w0_r[b]) + deg * b0_r[b], 0.0))
            out[...] = _dot(g1, w1_r[b]) + b1_r[b]

    return pl.pallas_call(
        body,
        out_shape=[jax.ShapeDtypeStruct((N, D), F32) for _ in range(3)],
    )(s_agg, deg16, w0, b0, w1, b1)


def _final(p2, q2, as2, h0, h1, h2, f_in, wsp, bsp, wc, bc, e8):
    def body(p_r, q_r, a_r, h0_r, h1_r, h2_r, f_r, wsp_r, bsp_r, wc_r, bc_r,
             e8_r, o_r):
        hgat = _bn(_dot(a_r[...], e8_r[...]) * p_r[...] + q_r[...])
        out = _dot(hgat, wc_r[0])
        for b, h_r in enumerate((h0_r, h1_r, h2_r)):
            out = out + _dot(_bn(jnp.maximum(h_r[...], 0.0)), wc_r[1 + b])
        sp = _bn(jnp.maximum(_dot(f_r[...], wsp_r[...]) + bsp_r[...], 0.0))
        o_r[...] = out + _dot(sp, wc_r[4]) + bc_r[...]

    return pl.pallas_call(
        body,
        out_shape=jax.ShapeDtypeStruct((N, NCLASS), F32),
    )(p2, q2, as2, h0, h1, h2, f_in, wsp, bsp, wc, bc, e8)


def kernel(f_in, edge_index, adj_vals, gat_Wself, gat_bself, gat_Wneigh,
           gat_bneigh, gat_aself, gat_aneigh, gcn_W, gcn_b, W_self_p,
           b_self_p, W_cls, b_cls):
    f_in = f_in.astype(F32)
    row3 = edge_index[0].reshape(NSUB * NCHUNK, 1, EC).astype(jnp.int32)
    col3 = edge_index[1].reshape(NSUB * NCHUNK, 1, EC).astype(jnp.int32)
    v3 = adj_vals.reshape(NSUB * NCHUNK, 1, EC).astype(F32)

    def agg(x):
        return _sc_agg(x, row3, col3, v3)[:N]

    eye = jnp.eye(HEADS, dtype=F32)
    e8 = jnp.repeat(eye, HD, axis=1)

    def cat_w(w):
        return w.transpose(1, 0, 2).reshape(D, D)

    def att_m(a):
        return (a[:, :, 0][:, :, None] * eye[:, None, :]).reshape(D, HEADS)

    fn1, u1, as1 = _gat_dense(
        f_in, cat_w(gat_Wself[0]), gat_bself[0].reshape(1, D),
        cat_w(gat_Wneigh[0]), gat_bneigh[0].reshape(1, D),
        att_m(gat_aself[0]), att_m(gat_aneigh[0]), e8)

    p1 = agg(fn1)
    q1 = agg(u1)
    s_agg = agg(f_in)
    deg16 = agg(jnp.ones((N, D), F32))[:, :HD]

    fn2, u2, as2 = _gat_mid(
        p1, q1, as1, cat_w(gat_Wself[1]), gat_bself[1].reshape(1, D),
        cat_w(gat_Wneigh[1]), gat_bneigh[1].reshape(1, D),
        att_m(gat_aself[1]), att_m(gat_aneigh[1]), e8)

    g0, g1, g2 = _gcn_mid(s_agg, deg16, gcn_W[:, 0],
                          gcn_b[:, 0].reshape(3, 1, D), gcn_W[:, 1],
                          gcn_b[:, 1].reshape(3, 1, D))

    p2 = agg(fn2)
    q2 = agg(u2)
    h0 = agg(g0)
    h1 = agg(g1)
    h2 = agg(g2)

    return _final(p2, q2, as2, h0, h1, h2, f_in, W_self_p,
                  b_self_p.reshape(1, D), W_cls.reshape(5, D, NCLASS),
                  b_cls.reshape(1, NCLASS), e8)


# speedup vs baseline: 21.6070x; 1.7437x over previous
import functools

import jax
import jax.numpy as jnp
from jax import lax
from jax.experimental import pallas as pl
from jax.experimental.pallas import tpu as pltpu
from jax.experimental.pallas import tpu_sc as plsc

N = 10000
E = 320000
D = 128
HEADS = 8
HD = 16
NCLASS = 8
F32 = jnp.float32

NSUB = 16
EPT = E // NSUB
EC = 80
NCHUNK = EPT // EC
NPAD = 10240
RPT = NPAD // NSUB
ZROWS = 64
LANES = 16

_MESH = plsc.VectorSubcoreMesh(core_axis_name="c", subcore_axis_name="s")


def _make_sc_stage(num_planes, passes0, passes1):
    assert len(passes0) == len(passes1)

    @functools.partial(
        pl.kernel, mesh=_MESH,
        out_type=jax.ShapeDtypeStruct((num_planes, NPAD, D), F32),
        scratch_types=[
            pltpu.VMEM_SHARED((NPAD, D), F32),
            pltpu.VMEM((1, EC), jnp.int32),
            pltpu.VMEM((1, EC), jnp.int32),
            pltpu.VMEM((1, EC), jnp.int32),
            pltpu.VMEM((1, EC), F32),
            pltpu.VMEM((EC, D), F32),
            pltpu.VMEM((ZROWS, D), F32),
        ],
    )
    def sc_stage(table, row_h, col_h, v_h, out, acc, rowc, colc, colc2, vc,
                 gbuf, zbuf):
        c = lax.axis_index("c")
        s = lax.axis_index("s")
        zero = jnp.zeros((LANES,), F32)

        @pl.loop(0, ZROWS)
        def _(r):
            zrow = zbuf.at[r]
            for k in range(D // LANES):
                zrow[pl.ds(LANES * k, LANES)] = zero

        for p0, p1 in zip(passes0, passes1):
            tid = jnp.where(c == 0, p0, p1).astype(jnp.int32)
            base = tid * N

            for i in range(RPT // ZROWS):
                pltpu.sync_copy(zbuf,
                                acc.at[pl.ds(s * RPT + i * ZROWS, ZROWS)])
            plsc.subcore_barrier()

            @pl.loop(0, NCHUNK)
            def _(j):
                g = s * NCHUNK + j
                pltpu.sync_copy(row_h.at[g], rowc)
                pltpu.sync_copy(col_h.at[g], colc)
                pltpu.sync_copy(v_h.at[g], vc)
                crow = colc.at[0]
                c2row = colc2.at[0]

                @pl.loop(0, EC // LANES)
                def _(t):
                    sl = pl.ds(LANES * t, LANES)
                    c2row[sl] = crow[sl] + base

                pltpu.sync_copy(table.at[colc2.at[0]], gbuf)
                vrow = vc.at[0]

                @pl.loop(0, EC // LANES)
                def _(t):
                    vt = vrow[pl.ds(LANES * t, LANES)]
                    for l in range(LANES):
                        vb = jnp.full((LANES,), vt[l], F32)
                        grow = gbuf.at[LANES * t + l]
                        for k in range(D // LANES):
                            sl = pl.ds(LANES * k, LANES)
                            grow[sl] = grow[sl] * vb

                pltpu.sync_copy(gbuf, acc.at[rowc.at[0]], add=True)

            plsc.subcore_barrier()
            pltpu.sync_copy(acc.at[pl.ds(s * RPT, RPT)],
                            out.at[tid].at[pl.ds(s * RPT, RPT)])
            plsc.subcore_barrier()

    return sc_stage


_sc_stage1 = _make_sc_stage(4, (0, 1), (2, 3))
_sc_stage2 = _make_sc_stage(5, (0, 1, 2), (3, 4, 4))


def _lrelu(x):
    return jnp.where(x > 0, x, 0.2 * x)


def _bn(x):
    m = jnp.mean(x, axis=0, keepdims=True)
    xc = x - m
    var = jnp.mean(xc * xc, axis=0, keepdims=True)
    return xc * lax.rsqrt(var + 1e-9)


def _dot(a, b):
    return jnp.dot(a, b, preferred_element_type=F32)


def _gat_dense(h, ws, bs, wn, bn_, ams, amn, e8):
    def body(h_r, ws_r, bs_r, wn_r, bn_r, ams_r, amn_r, e8_r, fn_o, u_o, as_o):
        hh = h_r[...]
        fs = jnp.maximum(_dot(hh, ws_r[...]) + bs_r[...], 0.0)
        fn = jnp.maximum(_dot(hh, wn_r[...]) + bn_r[...], 0.0)
        as_o[...] = _lrelu(_dot(fs, ams_r[...]))
        an = _lrelu(_dot(fs, amn_r[...]))
        fn_o[...] = fn
        u_o[...] = _dot(an, e8_r[...]) * fn

    return pl.pallas_call(
        body,
        out_shape=[jax.ShapeDtypeStruct((N, D), F32),
                   jax.ShapeDtypeStruct((N, D), F32),
                   jax.ShapeDtypeStruct((N, HEADS), F32)],
    )(h, ws, bs, wn, bn_, ams, amn, e8)


def _gat_mid(p1, q1, as1, ws, bs, wn, bn_, ams, amn, e8):
    def body(p_r, q_r, a_r, ws_r, bs_r, wn_r, bn_r, ams_r, amn_r, e8_r,
             fn_o, u_o, as_o):
        h1 = _bn(_dot(a_r[...], e8_r[...]) * p_r[...] + q_r[...])
        fs = jnp.maximum(_dot(h1, ws_r[...]) + bs_r[...], 0.0)
        fn = jnp.maximum(_dot(h1, wn_r[...]) + bn_r[...], 0.0)
        as_o[...] = _lrelu(_dot(fs, ams_r[...]))
        an = _lrelu(_dot(fs, amn_r[...]))
        fn_o[...] = fn
        u_o[...] = _dot(an, e8_r[...]) * fn

    return pl.pallas_call(
        body,
        out_shape=[jax.ShapeDtypeStruct((N, D), F32),
                   jax.ShapeDtypeStruct((N, D), F32),
                   jax.ShapeDtypeStruct((N, HEADS), F32)],
    )(p1, q1, as1, ws, bs, wn, bn_, ams, amn, e8)


def _gcn_mid(s_agg, deg16, w0, b0, w1, b1):
    def body(s_r, d_r, w0_r, b0_r, w1_r, b1_r, g0_o, g1_o, g2_o):
        ss = s_r[...]
        deg = d_r[:, 0:1]
        for b, out in enumerate((g0_o, g1_o, g2_o)):
            g1 = _bn(jnp.maximum(_dot(ss, w0_r[b]) + deg * b0_r[b], 0.0))
            out[...] = _dot(g1, w1_r[b]) + b1_r[b]

    return pl.pallas_call(
        body,
        out_shape=[jax.ShapeDtypeStruct((N, D), F32) for _ in range(3)],
    )(s_agg, deg16, w0, b0, w1, b1)


def _final(p2, q2, as2, h0, h1, h2, f_in, wsp, bsp, wc, bc, e8):
    def body(p_r, q_r, a_r, h0_r, h1_r, h2_r, f_r, wsp_r, bsp_r, wc_r, bc_r,
             e8_r, o_r):
        hgat = _bn(_dot(a_r[...], e8_r[...]) * p_r[...] + q_r[...])
        out = _dot(hgat, wc_r[0])
        for b, h_r in enumerate((h0_r, h1_r, h2_r)):
            out = out + _dot(_bn(jnp.maximum(h_r[...], 0.0)), wc_r[1 + b])
        sp = _bn(jnp.maximum(_dot(f_r[...], wsp_r[...]) + bsp_r[...], 0.0))
        o_r[...] = out + _dot(sp, wc_r[4]) + bc_r[...]

    return pl.pallas_call(
        body,
        out_shape=jax.ShapeDtypeStruct((N, NCLASS), F32),
    )(p2, q2, as2, h0, h1, h2, f_in, wsp, bsp, wc, bc, e8)


def kernel(f_in, edge_index, adj_vals, gat_Wself, gat_bself, gat_Wneigh,
           gat_bneigh, gat_aself, gat_aneigh, gcn_W, gcn_b, W_self_p,
           b_self_p, W_cls, b_cls):
    f_in = f_in.astype(F32)
    row3 = edge_index[0].reshape(NSUB * NCHUNK, 1, EC).astype(jnp.int32)
    col3 = edge_index[1].reshape(NSUB * NCHUNK, 1, EC).astype(jnp.int32)
    v3 = adj_vals.reshape(NSUB * NCHUNK, 1, EC).astype(F32)

    eye = jnp.eye(HEADS, dtype=F32)
    e8 = jnp.repeat(eye, HD, axis=1)

    def cat_w(w):
        return w.transpose(1, 0, 2).reshape(D, D)

    def att_m(a):
        return (a[:, :, 0][:, :, None] * eye[:, None, :]).reshape(D, HEADS)

    fn1, u1, as1 = _gat_dense(
        f_in, cat_w(gat_Wself[0]), gat_bself[0].reshape(1, D),
        cat_w(gat_Wneigh[0]), gat_bneigh[0].reshape(1, D),
        att_m(gat_aself[0]), att_m(gat_aneigh[0]), e8)

    t1 = jnp.concatenate([fn1, u1, f_in, jnp.ones((N, D), F32)], axis=0)
    o1 = _sc_stage1(t1, row3, col3, v3)
    p1, q1, s_agg = o1[0, :N], o1[1, :N], o1[2, :N]
    deg16 = o1[3, :N, :HD]

    fn2, u2, as2 = _gat_mid(
        p1, q1, as1, cat_w(gat_Wself[1]), gat_bself[1].reshape(1, D),
        cat_w(gat_Wneigh[1]), gat_bneigh[1].reshape(1, D),
        att_m(gat_aself[1]), att_m(gat_aneigh[1]), e8)

    g0, g1, g2 = _gcn_mid(s_agg, deg16, gcn_W[:, 0],
                          gcn_b[:, 0].reshape(3, 1, D), gcn_W[:, 1],
                          gcn_b[:, 1].reshape(3, 1, D))

    t2 = jnp.concatenate([fn2, u2, g0, g1, g2], axis=0)
    o2 = _sc_stage2(t2, row3, col3, v3)
    p2, q2, h0, h1, h2 = (o2[i, :N] for i in range(5))

    return _final(p2, q2, as2, h0, h1, h2, f_in, W_self_p,
                  b_self_p.reshape(1, D), W_cls.reshape(5, D, NCLASS),
                  b_cls.reshape(1, NCLASS), e8)


# speedup vs baseline: 59.1674x; 2.7383x over previous
import functools

import jax
import jax.numpy as jnp
from jax import lax
from jax.experimental import pallas as pl
from jax.experimental.pallas import tpu as pltpu
from jax.experimental.pallas import tpu_sc as plsc

N = 10000
E = 320000
D = 128
HEADS = 8
HD = 16
NCLASS = 8
F32 = jnp.float32

NSUB = 16
EPT = E // NSUB
EC = 80
NCHUNK = EPT // EC
G = 50
NGRP = NCHUNK // G
NPAD = 10240
RPT = NPAD // NSUB
ZROWS = 16
LANES = 16

_MESH = plsc.VectorSubcoreMesh(core_axis_name="c", subcore_axis_name="s")


def _make_sc_stage(num_planes, passes0, passes1):
    assert len(passes0) == len(passes1)

    @functools.partial(
        pl.kernel, mesh=_MESH,
        out_type=jax.ShapeDtypeStruct((num_planes, NPAD, D), F32),
        scratch_types=[
            pltpu.VMEM_SHARED((NPAD, D), F32),
            pltpu.VMEM((G, EC), jnp.int32),
            pltpu.VMEM((G, EC), jnp.int32),
            pltpu.VMEM((G, EC), F32),
            pltpu.VMEM((EC, D), F32),
            pltpu.VMEM((EC, D), F32),
            pltpu.VMEM((ZROWS, D), F32),
            pltpu.SemaphoreType.DMA,
            pltpu.SemaphoreType.DMA,
        ],
    )
    def sc_stage(table, row_h, col_h, v_h, out, acc, rowg, colg, vg,
                 gbuf0, gbuf1, zbuf, sem0, sem1):
        c = lax.axis_index("c")
        s = lax.axis_index("s")
        zero = jnp.zeros((LANES,), F32)
        bufs = (gbuf0, gbuf1)
        sems = (sem0, sem1)

        @pl.loop(0, ZROWS)
        def _(r):
            zrow = zbuf.at[r]
            for k in range(D // LANES):
                zrow[pl.ds(LANES * k, LANES)] = zero

        def gather(j, b):
            return pltpu.make_async_copy(table.at[colg.at[j]], bufs[b],
                                         sems[b])

        for p0, p1 in zip(passes0, passes1):
            tid = jnp.where(c == 0, p0, p1).astype(jnp.int32)
            base = tid * N

            for i in range(RPT // ZROWS):
                pltpu.sync_copy(zbuf,
                                acc.at[pl.ds(s * RPT + i * ZROWS, ZROWS)])
            plsc.subcore_barrier()

            @pl.loop(0, NGRP)
            def _(gi):
                pltpu.sync_copy(row_h.at[s].at[gi], rowg)
                pltpu.sync_copy(col_h.at[s].at[gi], colg)
                pltpu.sync_copy(v_h.at[s].at[gi], vg)

                @pl.loop(0, G)
                def _(j):
                    crow = colg.at[j]

                    @pl.loop(0, EC // LANES)
                    def _(t):
                        sl = pl.ds(LANES * t, LANES)
                        crow[sl] = crow[sl] + base

                gather(0, 0).start()
                gather(1, 1).start()

                @pl.loop(0, G // 2)
                def _(jj):
                    for b in range(2):
                        j = 2 * jj + b
                        gather(j, b).wait()
                        vrow = vg.at[j]

                        @pl.loop(0, EC // LANES)
                        def _(t):
                            vt = vrow[pl.ds(LANES * t, LANES)]
                            for l in range(LANES):
                                vb = jnp.full((LANES,), vt[l], F32)
                                grow = bufs[b].at[LANES * t + l]
                                for k in range(D // LANES):
                                    sl = pl.ds(LANES * k, LANES)
                                    grow[sl] = grow[sl] * vb

                        pltpu.sync_copy(bufs[b], acc.at[rowg.at[j]],
                                        add=True)

                        @pl.when(jj < G // 2 - 1)
                        def _():
                            gather(j + 2, b).start()

            plsc.subcore_barrier()
            pltpu.sync_copy(acc.at[pl.ds(s * RPT, RPT)],
                            out.at[tid].at[pl.ds(s * RPT, RPT)])
            plsc.subcore_barrier()

    return sc_stage


_sc_stage1 = _make_sc_stage(4, (0, 1), (2, 3))
_sc_stage2 = _make_sc_stage(5, (0, 1, 2), (3, 4, 4))


def _lrelu(x):
    return jnp.where(x > 0, x, 0.2 * x)


def _bn(x):
    m = jnp.mean(x, axis=0, keepdims=True)
    xc = x - m
    var = jnp.mean(xc * xc, axis=0, keepdims=True)
    return xc * lax.rsqrt(var + 1e-9)


def _dot(a, b):
    return jnp.dot(a, b, preferred_element_type=F32)


def _gat_dense(h, ws, bs, wn, bn_, ams, amn, e8):
    def body(h_r, ws_r, bs_r, wn_r, bn_r, ams_r, amn_r, e8_r, fn_o, u_o, as_o):
        hh = h_r[...]
        fs = jnp.maximum(_dot(hh, ws_r[...]) + bs_r[...], 0.0)
        fn = jnp.maximum(_dot(hh, wn_r[...]) + bn_r[...], 0.0)
        as_o[...] = _lrelu(_dot(fs, ams_r[...]))
        an = _lrelu(_dot(fs, amn_r[...]))
        fn_o[...] = fn
        u_o[...] = _dot(an, e8_r[...]) * fn

    return pl.pallas_call(
        body,
        out_shape=[jax.ShapeDtypeStruct((N, D), F32),
                   jax.ShapeDtypeStruct((N, D), F32),
                   jax.ShapeDtypeStruct((N, HEADS), F32)],
    )(h, ws, bs, wn, bn_, ams, amn, e8)


def _gat_mid(p1, q1, as1, ws, bs, wn, bn_, ams, amn, e8):
    def body(p_r, q_r, a_r, ws_r, bs_r, wn_r, bn_r, ams_r, amn_r, e8_r,
             fn_o, u_o, as_o):
        h1 = _bn(_dot(a_r[...], e8_r[...]) * p_r[...] + q_r[...])
        fs = jnp.maximum(_dot(h1, ws_r[...]) + bs_r[...], 0.0)
        fn = jnp.maximum(_dot(h1, wn_r[...]) + bn_r[...], 0.0)
        as_o[...] = _lrelu(_dot(fs, ams_r[...]))
        an = _lrelu(_dot(fs, amn_r[...]))
        fn_o[...] = fn
        u_o[...] = _dot(an, e8_r[...]) * fn

    return pl.pallas_call(
        body,
        out_shape=[jax.ShapeDtypeStruct((N, D), F32),
                   jax.ShapeDtypeStruct((N, D), F32),
                   jax.ShapeDtypeStruct((N, HEADS), F32)],
    )(p1, q1, as1, ws, bs, wn, bn_, ams, amn, e8)


def _gcn_mid(s_agg, deg16, w0, b0, w1, b1):
    def body(s_r, d_r, w0_r, b0_r, w1_r, b1_r, g0_o, g1_o, g2_o):
        ss = s_r[...]
        deg = d_r[:, 0:1]
        for b, out in enumerate((g0_o, g1_o, g2_o)):
            g1 = _bn(jnp.maximum(_dot(ss, w0_r[b]) + deg * b0_r[b], 0.0))
            out[...] = _dot(g1, w1_r[b]) + b1_r[b]

    return pl.pallas_call(
        body,
        out_shape=[jax.ShapeDtypeStruct((N, D), F32) for _ in range(3)],
    )(s_agg, deg16, w0, b0, w1, b1)


def _final(p2, q2, as2, h0, h1, h2, f_in, wsp, bsp, wc, bc, e8):
    def body(p_r, q_r, a_r, h0_r, h1_r, h2_r, f_r, wsp_r, bsp_r, wc_r, bc_r,
             e8_r, o_r):
        hgat = _bn(_dot(a_r[...], e8_r[...]) * p_r[...] + q_r[...])
        out = _dot(hgat, wc_r[0])
        for b, h_r in enumerate((h0_r, h1_r, h2_r)):
            out = out + _dot(_bn(jnp.maximum(h_r[...], 0.0)), wc_r[1 + b])
        sp = _bn(jnp.maximum(_dot(f_r[...], wsp_r[...]) + bsp_r[...], 0.0))
        o_r[...] = out + _dot(sp, wc_r[4]) + bc_r[...]

    return pl.pallas_call(
        body,
        out_shape=jax.ShapeDtypeStruct((N, NCLASS), F32),
    )(p2, q2, as2, h0, h1, h2, f_in, wsp, bsp, wc, bc, e8)


def kernel(f_in, edge_index, adj_vals, gat_Wself, gat_bself, gat_Wneigh,
           gat_bneigh, gat_aself, gat_aneigh, gcn_W, gcn_b, W_self_p,
           b_self_p, W_cls, b_cls):
    f_in = f_in.astype(F32)
    row3 = edge_index[0].reshape(NSUB, NGRP, G, EC).astype(jnp.int32)
    col3 = edge_index[1].reshape(NSUB, NGRP, G, EC).astype(jnp.int32)
    v3 = adj_vals.reshape(NSUB, NGRP, G, EC).astype(F32)

    eye = jnp.eye(HEADS, dtype=F32)
    e8 = jnp.repeat(eye, HD, axis=1)

    def cat_w(w):
        return w.transpose(1, 0, 2).reshape(D, D)

    def att_m(a):
        return (a[:, :, 0][:, :, None] * eye[:, None, :]).reshape(D, HEADS)

    fn1, u1, as1 = _gat_dense(
        f_in, cat_w(gat_Wself[0]), gat_bself[0].reshape(1, D),
        cat_w(gat_Wneigh[0]), gat_bneigh[0].reshape(1, D),
        att_m(gat_aself[0]), att_m(gat_aneigh[0]), e8)

    t1 = jnp.concatenate([fn1, u1, f_in, jnp.ones((N, D), F32)], axis=0)
    o1 = _sc_stage1(t1, row3, col3, v3)
    p1, q1, s_agg = o1[0, :N], o1[1, :N], o1[2, :N]
    deg16 = o1[3, :N, :HD]

    fn2, u2, as2 = _gat_mid(
        p1, q1, as1, cat_w(gat_Wself[1]), gat_bself[1].reshape(1, D),
        cat_w(gat_Wneigh[1]), gat_bneigh[1].reshape(1, D),
        att_m(gat_aself[1]), att_m(gat_aneigh[1]), e8)

    g0, g1, g2 = _gcn_mid(s_agg, deg16, gcn_W[:, 0],
                          gcn_b[:, 0].reshape(3, 1, D), gcn_W[:, 1],
                          gcn_b[:, 1].reshape(3, 1, D))

    t2 = jnp.concatenate([fn2, u2, g0, g1, g2], axis=0)
    o2 = _sc_stage2(t2, row3, col3, v3)
    p2, q2, h0, h1, h2 = (o2[i, :N] for i in range(5))

    return _final(p2, q2, as2, h0, h1, h2, f_in, W_self_p,
                  b_self_p.reshape(1, D), W_cls.reshape(5, D, NCLASS),
                  b_cls.reshape(1, NCLASS), e8)
